# baseline (speedup 1.0000x reference)
import jax
import jax.numpy as jnp
from jax import lax
from jax.experimental import pallas as pl
from jax.experimental.pallas import tpu as pltpu
from jax.experimental.pallas import tpu_sc as plsc

_B, _M, _N = 4, 4096, 4096
_NNZ = 167772
_NC, _NS, _L = 2, 16, 16
_NW = _NC * _NS
_TPG = _NW // _B
_CH = 20976
_CH7 = _NNZ - 7 * _CH
_T7FULL = (_CH7 // _L) * _L
_T7REM = _CH7 - _T7FULL
_Q0 = 5248
_SLICE = _M // _TPG
_UNROLL = 8

_W_PRIMAL, _W_DUAL, _W_STAT, _W_COMP = 0.1, 0.1, 0.6, 0.2


def _sc_kkt(x_hbm, lam_hbm, vals_hbm, rows_hbm, cols_hbm, b_hbm, c_hbm,
            tv_hbm, tr_hbm, tk_hbm,
            out_hbm,
            vals_v, rows_v, cols_v, tv_v, tr_v, tk_v, x_v, lam_v, ax_v, atl_v,
            bufa_v, bufb_v, b_v, c_v, outv,
            part_ax, part_atl, sh_x, sh_lam, sem_a, sem_b, sem_x):
    c = lax.axis_index("c")
    s = lax.axis_index("s")
    p = c * 2 + s // _TPG
    j = s % _TPG
    g0 = (s // _TPG) * _TPG
    wid = c * _NS + s
    pi = s // _TPG
    base = j * _CH
    scope = jax.named_scope

    @pl.when(j == 0)
    def _stage_xl():
        pltpu.async_copy(x_hbm.at[pl.ds(p * _N, _N)], sh_x.at[pi], sem_x)
        pltpu.async_copy(lam_hbm.at[pl.ds(p * _M, _M)], sh_lam.at[pi], sem_x)

    cps_bc = [
        pltpu.async_copy(b_hbm.at[p, pl.ds(j * _SLICE, _SLICE)], b_v, sem_x),
        pltpu.async_copy(c_hbm.at[p, pl.ds(j * _SLICE, _SLICE)], c_v, sem_x),
    ]

    def fire3(off, ln, csem):
        dst = pl.ds(off, ln)
        return [
            pltpu.async_copy(vals_hbm.at[p, pl.ds(base + off, ln)],
                             vals_v.at[dst], csem),
            pltpu.async_copy(rows_hbm.at[p, pl.ds(base + off, ln)],
                             rows_v.at[dst], csem),
            pltpu.async_copy(cols_hbm.at[p, pl.ds(base + off, ln)],
                             cols_v.at[dst], csem),
        ]

    def drain3(off, ln, csem):
        dst = pl.ds(off, ln)
        pltpu.make_async_copy(vals_hbm.at[p, pl.ds(base + off, ln)],
                              vals_v.at[dst], csem).wait()
        pltpu.make_async_copy(rows_hbm.at[p, pl.ds(base + off, ln)],
                              rows_v.at[dst], csem).wait()
        pltpu.make_async_copy(cols_hbm.at[p, pl.ds(base + off, ln)],
                              cols_v.at[dst], csem).wait()

    cps_q0 = fire3(0, _Q0, sem_a)

    zero16 = jnp.zeros((_L,), jnp.float32)

    def zero_body(off):
        ax_v[pl.ds(off, _L)] = zero16
        atl_v[pl.ds(off, _L)] = zero16

    with scope("p0_zero"):
        plsc.parallel_loop(0, _M, _L, unroll=8)(zero_body)

    with scope("p0_bcast"):
        @pl.when(j == 0)
        def _wait_xl():
            pltpu.make_async_copy(x_hbm.at[pl.ds(p * _N, _N)],
                                  sh_x.at[pi], sem_x).wait()
            pltpu.make_async_copy(lam_hbm.at[pl.ds(p * _M, _M)],
                                  sh_lam.at[pi], sem_x).wait()
        plsc.subcore_barrier()
        cp1 = pltpu.async_copy(sh_x.at[pi], x_v, sem_x)
        cp2 = pltpu.async_copy(sh_lam.at[pi], lam_v, sem_x)
        cp1.wait()
        cp2.wait()

    with scope("p0_wait"):
        for cp in cps_q0:
            cp.wait()

    @pl.when(j < _TPG - 1)
    def _fire_rest():
        fire3(_Q0, _CH - _Q0, sem_b)

    @pl.when(j == _TPG - 1)
    def _fire_rest7():
        fire3(_Q0, _T7FULL - _Q0, sem_b)
        pltpu.async_copy(tv_hbm.at[p], tv_v, sem_b)
        pltpu.async_copy(tr_hbm.at[p], tr_v, sem_b)
        pltpu.async_copy(tk_hbm.at[p], tk_v, sem_b)

    def group(off):
        v16 = vals_v[pl.ds(off, _L)]
        r16 = rows_v[pl.ds(off, _L)]
        k16 = cols_v[pl.ds(off, _L)]
        xg = plsc.load_gather(x_v, [k16])
        plsc.addupdate_scatter(ax_v, [r16], v16 * xg)
        lg = plsc.load_gather(lam_v, [r16])
        plsc.addupdate_scatter(atl_v, [k16], v16 * lg)

    with scope("p1_spmm"):
        plsc.parallel_loop(0, _Q0, _L, unroll=_UNROLL)(group)

        @pl.when(j < _TPG - 1)
        def _drain_rest():
            drain3(_Q0, _CH - _Q0, sem_b)

        @pl.when(j == _TPG - 1)
        def _drain_rest7():
            drain3(_Q0, _T7FULL - _Q0, sem_b)
            pltpu.make_async_copy(tv_hbm.at[p], tv_v, sem_b).wait()
            pltpu.make_async_copy(tr_hbm.at[p], tr_v, sem_b).wait()
            pltpu.make_async_copy(tk_hbm.at[p], tk_v, sem_b).wait()

        plsc.parallel_loop(_Q0, _T7FULL, _L, unroll=_UNROLL)(group)

        @pl.when(j < _TPG - 1)
        def _tail_full():
            for g in range((_CH - _T7FULL) // _L):
                group(_T7FULL + g * _L)

        @pl.when(j == _TPG - 1)
        def _tail_masked():
            lanes = lax.iota(jnp.int32, _L)
            live = lanes >= (_L - _T7REM)
            v16 = jnp.where(live, tv_v[...], 0.0)
            r16 = jnp.where(live, tr_v[...], 0)
            k16 = jnp.where(live, tk_v[...], 0)
            xg = plsc.load_gather(x_v, [k16])
            plsc.addupdate_scatter(ax_v, [r16], v16 * xg)
            lg = plsc.load_gather(lam_v, [r16])
            plsc.addupdate_scatter(atl_v, [k16], v16 * lg)

    with scope("p2_pub"):
        cp1 = pltpu.async_copy(ax_v, part_ax.at[s], sem_x)
        cp2 = pltpu.async_copy(atl_v, part_atl.at[s], sem_x)
        cp1.wait()
        cp2.wait()
        plsc.subcore_barrier()

    off = j * _SLICE
    cps = [
        pltpu.async_copy(
            part_ax.at[pl.ds(g0, _TPG), pl.ds(off, _SLICE)], bufa_v, sem_x),
        pltpu.async_copy(
            part_atl.at[pl.ds(g0, _TPG), pl.ds(off, _SLICE)], bufb_v, sem_x),
    ]
    with scope("p2_pull"):
        for cp in cps:
            cp.wait()

    def loss_body(t, acc):
        acc_p, acc_d, acc_s, acc_c = acc
        ds16 = pl.ds(t * _L, _L)
        ax16 = bufa_v[0, ds16]
        atl16 = bufb_v[0, ds16]
        for k in range(1, _TPG):
            ax16 = ax16 + bufa_v[k, ds16]
            atl16 = atl16 + bufb_v[k, ds16]
        b16 = b_v[ds16]
        c16 = c_v[ds16]
        lam16 = lam_v[pl.ds(off + t * _L, _L)]
        axmb = ax16 - b16
        relu_axmb = jnp.maximum(axmb, 0.0)
        relu_nlam = jnp.maximum(-lam16, 0.0)
        st = atl16 + c16
        cp16 = lam16 * axmb
        return (acc_p + relu_axmb * relu_axmb,
                acc_d + relu_nlam * relu_nlam,
                acc_s + st * st,
                acc_c + cp16 * cp16)

    acc0 = (zero16, zero16, zero16, zero16)
    for cp in cps_bc:
        cp.wait()
    with scope("p3_loss"):
        acc_p, acc_d, acc_s, acc_c = lax.fori_loop(
            0, _SLICE // _L, loss_body, acc0)

    scale = 1.0 / (_M * _B)
    outv[...] = (_W_PRIMAL * acc_p + _W_DUAL * acc_d
                 + _W_STAT * acc_s + _W_COMP * acc_c) * scale
    pltpu.async_copy(outv, out_hbm.at[wid], sem_x).wait()


@jax.jit
def _run(x_hat, lam_hat, vals, rows, cols, b_pad, c_pad, tv, tr, tk):
    mesh = plsc.VectorSubcoreMesh(core_axis_name="c", subcore_axis_name="s",
                                  num_cores=_NC, num_subcores=_NS)
    kern = pl.kernel(
        _sc_kkt,
        out_type=jax.ShapeDtypeStruct((_NW, _L), jnp.float32),
        mesh=mesh,
        scratch_types=[
            pltpu.VMEM((_CH,), jnp.float32),
            pltpu.VMEM((_CH,), jnp.int32),
            pltpu.VMEM((_CH,), jnp.int32),
            pltpu.VMEM((_L,), jnp.float32),
            pltpu.VMEM((_L,), jnp.int32),
            pltpu.VMEM((_L,), jnp.int32),
            pltpu.VMEM((_N,), jnp.float32),
            pltpu.VMEM((_M,), jnp.float32),
            pltpu.VMEM((_M,), jnp.float32),
            pltpu.VMEM((_N,), jnp.float32),
            pltpu.VMEM((_TPG, _SLICE), jnp.float32),
            pltpu.VMEM((_TPG, _SLICE), jnp.float32),
            pltpu.VMEM((_SLICE,), jnp.float32),
            pltpu.VMEM((_SLICE,), jnp.float32),
            pltpu.VMEM((_L,), jnp.float32),
            pltpu.VMEM_SHARED((_NS, _M), jnp.float32),
            pltpu.VMEM_SHARED((_NS, _N), jnp.float32),
            pltpu.VMEM_SHARED((2, _N), jnp.float32),
            pltpu.VMEM_SHARED((2, _M), jnp.float32),
            pltpu.SemaphoreType.DMA,
            pltpu.SemaphoreType.DMA,
            pltpu.SemaphoreType.DMA,
        ],
        compiler_params=pltpu.CompilerParams(needs_layout_passes=False,
                                             use_tc_tiling_on_sc=False),
    )
    out = kern(x_hat, lam_hat, vals, rows, cols, b_pad, c_pad, tv, tr, tk)
    return jnp.sum(out)


def kernel(x_hat, lam_hat, A_vals, A_rows, A_cols, b_pad, c_pad):
    rows = A_rows.astype(jnp.int32)
    cols = A_cols.astype(jnp.int32)
    return _run(x_hat, lam_hat, A_vals, rows, cols, b_pad, c_pad,
                A_vals[:, _NNZ - _L:], rows[:, _NNZ - _L:],
                cols[:, _NNZ - _L:])


# speedup vs baseline: 1.1463x; 1.1463x over previous
import jax
import jax.numpy as jnp
from jax import lax
from jax.experimental import pallas as pl
from jax.experimental.pallas import tpu as pltpu
from jax.experimental.pallas import tpu_sc as plsc

_B, _M, _N = 4, 4096, 4096
_NNZ = 167772
_NC, _NS, _L = 2, 16, 16
_NW = _NC * _NS
_TPG = _NW // _B
_CH = 20976
_CH7 = _NNZ - 7 * _CH
_T7FULL = (_CH7 // _L) * _L
_T7REM = _CH7 - _T7FULL
_Q0 = 5248
_SLICE = _M // _TPG
_UNROLL = 8

_W_PRIMAL, _W_DUAL, _W_STAT, _W_COMP = 0.1, 0.1, 0.6, 0.2


def _sc_kkt(x_hbm, lam_hbm, vals_hbm, rc_hbm, b_hbm, c_hbm,
            hv_hbm, hrc_hbm, tv_hbm, trc_hbm,
            out_hbm,
            vals_v, rc_v, hv_v, hrc_v, tv_v, trc_v, x_v, lam_v, ax_v, atl_v,
            bufa_v, bufb_v, b_v, c_v, outv,
            part_ax, part_atl, sh_x, sh_lam, sem_a, sem_b, sem_x):
    c = lax.axis_index("c")
    s = lax.axis_index("s")
    p = c * 2 + s // _TPG
    j = s % _TPG
    g0 = (s // _TPG) * _TPG
    wid = c * _NS + s
    pi = s // _TPG
    skip = (p & 1) * 4
    base = pl.multiple_of(p * _NNZ + skip + j * _CH, 8)
    scope = jax.named_scope

    @pl.when(j == 0)
    def _stage_xl():
        pltpu.async_copy(x_hbm.at[pl.ds(p * _N, _N)], sh_x.at[pi], sem_x)
        pltpu.async_copy(lam_hbm.at[pl.ds(p * _M, _M)], sh_lam.at[pi], sem_x)

    cps_bc = [
        pltpu.async_copy(b_hbm.at[pl.ds(p * _M + j * _SLICE, _SLICE)],
                         b_v, sem_x),
        pltpu.async_copy(c_hbm.at[pl.ds(p * _N + j * _SLICE, _SLICE)],
                         c_v, sem_x),
    ]

    def fire2(off, ln, csem):
        dst = pl.ds(off, ln)
        return [
            pltpu.async_copy(vals_hbm.at[pl.ds(base + off, ln)],
                             vals_v.at[dst], csem),
            pltpu.async_copy(rc_hbm.at[pl.ds(base + off, ln)],
                             rc_v.at[dst], csem),
        ]

    def drain2(off, ln, csem):
        dst = pl.ds(off, ln)
        pltpu.make_async_copy(vals_hbm.at[pl.ds(base + off, ln)],
                              vals_v.at[dst], csem).wait()
        pltpu.make_async_copy(rc_hbm.at[pl.ds(base + off, ln)],
                              rc_v.at[dst], csem).wait()

    cps_q0 = fire2(0, _Q0, sem_a)

    zero16 = jnp.zeros((_L,), jnp.float32)

    def zero_body(off):
        ax_v[pl.ds(off, _L)] = zero16
        atl_v[pl.ds(off, _L)] = zero16

    with scope("p0_zero"):
        plsc.parallel_loop(0, _M, _L, unroll=8)(zero_body)

    with scope("p0_bcast"):
        @pl.when(j == 0)
        def _wait_xl():
            pltpu.make_async_copy(x_hbm.at[pl.ds(p * _N, _N)],
                                  sh_x.at[pi], sem_x).wait()
            pltpu.make_async_copy(lam_hbm.at[pl.ds(p * _M, _M)],
                                  sh_lam.at[pi], sem_x).wait()
        plsc.subcore_barrier()
        cp1 = pltpu.async_copy(sh_x.at[pi], x_v, sem_x)
        cp2 = pltpu.async_copy(sh_lam.at[pi], lam_v, sem_x)
        cp1.wait()
        cp2.wait()

    with scope("p0_wait"):
        for cp in cps_q0:
            cp.wait()

    @pl.when(j < _TPG - 1)
    def _fire_rest():
        fire2(_Q0, _CH - _Q0, sem_b)

    @pl.when(j == _TPG - 1)
    def _fire_rest7():
        fire2(_Q0, _T7FULL - _Q0, sem_b)
        pltpu.async_copy(tv_hbm.at[p], tv_v, sem_b)
        pltpu.async_copy(trc_hbm.at[p], trc_v, sem_b)

    @pl.when(j == 0)
    def _fire_head():
        pltpu.async_copy(hv_hbm.at[p], hv_v, sem_b)
        pltpu.async_copy(hrc_hbm.at[p], hrc_v, sem_b)

    def group(off):
        v16 = vals_v[pl.ds(off, _L)]
        rc16 = rc_v[pl.ds(off, _L)]
        r16 = rc16 & 0xFFFF
        k16 = lax.shift_right_logical(rc16, 16)
        xg = plsc.load_gather(x_v, [k16])
        plsc.addupdate_scatter(ax_v, [r16], v16 * xg)
        lg = plsc.load_gather(lam_v, [r16])
        plsc.addupdate_scatter(atl_v, [k16], v16 * lg)

    def masked_group(v_ref, rc_ref, live):
        v16 = jnp.where(live, v_ref[...], 0.0)
        rc16 = jnp.where(live, rc_ref[...], 0)
        r16 = rc16 & 0xFFFF
        k16 = lax.shift_right_logical(rc16, 16)
        xg = plsc.load_gather(x_v, [k16])
        plsc.addupdate_scatter(ax_v, [r16], v16 * xg)
        lg = plsc.load_gather(lam_v, [r16])
        plsc.addupdate_scatter(atl_v, [k16], v16 * lg)

    with scope("p1_spmm"):
        plsc.parallel_loop(0, _Q0, _L, unroll=_UNROLL)(group)

        @pl.when(j < _TPG - 1)
        def _drain_rest():
            drain2(_Q0, _CH - _Q0, sem_b)

        @pl.when(j == _TPG - 1)
        def _drain_rest7():
            drain2(_Q0, _T7FULL - _Q0, sem_b)
            pltpu.make_async_copy(tv_hbm.at[p], tv_v, sem_b).wait()
            pltpu.make_async_copy(trc_hbm.at[p], trc_v, sem_b).wait()

        @pl.when(j == 0)
        def _drain_head():
            pltpu.make_async_copy(hv_hbm.at[p], hv_v, sem_b).wait()
            pltpu.make_async_copy(hrc_hbm.at[p], hrc_v, sem_b).wait()

        plsc.parallel_loop(_Q0, _T7FULL, _L, unroll=_UNROLL)(group)

        lanes = lax.iota(jnp.int32, _L)

        @pl.when(j < _TPG - 1)
        def _tail_full():
            for g in range((_CH - _T7FULL) // _L):
                group(_T7FULL + g * _L)

        @pl.when(j == 0)
        def _head_masked():
            masked_group(hv_v, hrc_v, lanes < skip)

        @pl.when(j == _TPG - 1)
        def _tail_masked():
            masked_group(tv_v, trc_v, lanes >= 4 + skip)

    with scope("p2_pub"):
        cp1 = pltpu.async_copy(ax_v, part_ax.at[s], sem_x)
        cp2 = pltpu.async_copy(atl_v, part_atl.at[s], sem_x)
        cp1.wait()
        cp2.wait()
        plsc.subcore_barrier()

    off = j * _SLICE
    cps = [
        pltpu.async_copy(
            part_ax.at[pl.ds(g0, _TPG), pl.ds(off, _SLICE)], bufa_v, sem_x),
        pltpu.async_copy(
            part_atl.at[pl.ds(g0, _TPG), pl.ds(off, _SLICE)], bufb_v, sem_x),
    ]
    with scope("p2_pull"):
        for cp in cps:
            cp.wait()

    def loss_body(t, acc):
        acc_p, acc_d, acc_s, acc_c = acc
        ds16 = pl.ds(t * _L, _L)
        ax16 = bufa_v[0, ds16]
        atl16 = bufb_v[0, ds16]
        for k in range(1, _TPG):
            ax16 = ax16 + bufa_v[k, ds16]
            atl16 = atl16 + bufb_v[k, ds16]
        b16 = b_v[ds16]
        c16 = c_v[ds16]
        lam16 = lam_v[pl.ds(off + t * _L, _L)]
        axmb = ax16 - b16
        relu_axmb = jnp.maximum(axmb, 0.0)
        relu_nlam = jnp.maximum(-lam16, 0.0)
        st = atl16 + c16
        cp16 = lam16 * axmb
        return (acc_p + relu_axmb * relu_axmb,
                acc_d + relu_nlam * relu_nlam,
                acc_s + st * st,
                acc_c + cp16 * cp16)

    acc0 = (zero16, zero16, zero16, zero16)
    for cp in cps_bc:
        cp.wait()
    with scope("p3_loss"):
        acc_p, acc_d, acc_s, acc_c = lax.fori_loop(
            0, _SLICE // _L, loss_body, acc0)

    scale = 1.0 / (_M * _B)
    outv[...] = (_W_PRIMAL * acc_p + _W_DUAL * acc_d
                 + _W_STAT * acc_s + _W_COMP * acc_c) * scale
    pltpu.async_copy(outv, out_hbm.at[wid], sem_x).wait()


@jax.jit
def _run(x_hat, lam_hat, vals_f, rc_f, b_f, c_f, hv, hrc, tv, trc):
    mesh = plsc.VectorSubcoreMesh(core_axis_name="c", subcore_axis_name="s",
                                  num_cores=_NC, num_subcores=_NS)
    kern = pl.kernel(
        _sc_kkt,
        out_type=jax.ShapeDtypeStruct((_NW, _L), jnp.float32),
        mesh=mesh,
        scratch_types=[
            pltpu.VMEM((_CH,), jnp.float32),
            pltpu.VMEM((_CH,), jnp.int32),
            pltpu.VMEM((_L,), jnp.float32),
            pltpu.VMEM((_L,), jnp.int32),
            pltpu.VMEM((_L,), jnp.float32),
            pltpu.VMEM((_L,), jnp.int32),
            pltpu.VMEM((_N,), jnp.float32),
            pltpu.VMEM((_M,), jnp.float32),
            pltpu.VMEM((_M,), jnp.float32),
            pltpu.VMEM((_N,), jnp.float32),
            pltpu.VMEM((_TPG, _SLICE), jnp.float32),
            pltpu.VMEM((_TPG, _SLICE), jnp.float32),
            pltpu.VMEM((_SLICE,), jnp.float32),
            pltpu.VMEM((_SLICE,), jnp.float32),
            pltpu.VMEM((_L,), jnp.float32),
            pltpu.VMEM_SHARED((_NS, _M), jnp.float32),
            pltpu.VMEM_SHARED((_NS, _N), jnp.float32),
            pltpu.VMEM_SHARED((2, _N), jnp.float32),
            pltpu.VMEM_SHARED((2, _M), jnp.float32),
            pltpu.SemaphoreType.DMA,
            pltpu.SemaphoreType.DMA,
            pltpu.SemaphoreType.DMA,
        ],
        compiler_params=pltpu.CompilerParams(needs_layout_passes=False,
                                             use_tc_tiling_on_sc=False),
    )
    out = kern(x_hat, lam_hat, vals_f, rc_f, b_f, c_f, hv, hrc, tv, trc)
    return jnp.sum(out)


def kernel(x_hat, lam_hat, A_vals, A_rows, A_cols, b_pad, c_pad):
    rc = A_rows.astype(jnp.int32) | (A_cols.astype(jnp.int32) << 16)
    return _run(x_hat, lam_hat,
                A_vals.reshape(-1), rc.reshape(-1),
                b_pad.reshape(-1), c_pad.reshape(-1),
                A_vals[:, :_L], rc[:, :_L],
                A_vals[:, _NNZ - _L:], rc[:, _NNZ - _L:])


# speedup vs baseline: 1.2297x; 1.0728x over previous
import jax
import jax.numpy as jnp
from jax import lax
from jax.experimental import pallas as pl
from jax.experimental.pallas import tpu as pltpu
from jax.experimental.pallas import tpu_sc as plsc

_B, _M, _N = 4, 4096, 4096
_NNZ = 167772
_NC, _NS, _L = 2, 16, 16
_NW = _NC * _NS
_TPG = _NW // _B
_CH = 20992
_NNZ_PAD = _CH * _TPG
_NSPLIT = 4
_SUB = _CH // _NSPLIT
_SLICE = _M // _TPG
_UNROLL = 8

_W_PRIMAL, _W_DUAL, _W_STAT, _W_COMP = 0.1, 0.1, 0.6, 0.2


def _sc_kkt(x_hbm, lam_hbm, vals_hbm, rc_hbm, b_hbm, c_hbm,
            out_hbm,
            vals_v, rc_v, x_v, lam_v, ax_v, atl_v,
            bufa_v, bufb_v, b_v, c_v, outv,
            part_ax, part_atl, sh_x, sh_lam, sem_a, sem_b, sem_x):
    c = lax.axis_index("c")
    s = lax.axis_index("s")
    p = c * 2 + s // _TPG
    j = s % _TPG
    g0 = (s // _TPG) * _TPG
    wid = c * _NS + s
    pi = s // _TPG
    nz_base = p * _NNZ_PAD + j * _CH
    scope = jax.named_scope

    @pl.when(j == 0)
    def _stage_xl():
        pltpu.async_copy(x_hbm.at[pl.ds(p * _N, _N)], sh_x.at[pi], sem_x)
        pltpu.async_copy(lam_hbm.at[pl.ds(p * _M, _M)], sh_lam.at[pi], sem_x)

    cps_bc = [
        pltpu.async_copy(b_hbm.at[pl.ds(p * _M + j * _SLICE, _SLICE)], b_v, sem_x),
        pltpu.async_copy(c_hbm.at[pl.ds(p * _N + j * _SLICE, _SLICE)], c_v, sem_x),
    ]

    def fire(q, csem):
        base = nz_base + q * _SUB
        dst = pl.ds(q * _SUB, _SUB)
        return [
            pltpu.async_copy(vals_hbm.at[pl.ds(base, _SUB)],
                             vals_v.at[dst], csem),
            pltpu.async_copy(rc_hbm.at[pl.ds(base, _SUB)],
                             rc_v.at[dst], csem),
        ]

    cps_q0 = fire(0, sem_a)

    zero16 = jnp.zeros((_L,), jnp.float32)

    def zero_body(off):
        ax_v[pl.ds(off, _L)] = zero16
        atl_v[pl.ds(off, _L)] = zero16

    with jax.named_scope("p0_zero"):
        plsc.parallel_loop(0, _M, _L, unroll=8)(zero_body)

    with jax.named_scope("p0_bcast"):
        @pl.when(j == 0)
        def _wait_xl():
            pltpu.make_async_copy(x_hbm.at[pl.ds(p * _N, _N)],
                                  sh_x.at[pi], sem_x).wait()
            pltpu.make_async_copy(lam_hbm.at[pl.ds(p * _M, _M)],
                                  sh_lam.at[pi], sem_x).wait()
        plsc.subcore_barrier()
        cp1 = pltpu.async_copy(sh_x.at[pi], x_v, sem_x)
        cp2 = pltpu.async_copy(sh_lam.at[pi], lam_v, sem_x)
        cp1.wait()
        cp2.wait()

    with jax.named_scope("p0_wait"):
        for cp in cps_q0:
            cp.wait()

    cps_rest = []
    for q in range(1, _NSPLIT):
        cps_rest += fire(q, sem_b)

    def nnz_body(off):
        v16 = vals_v[pl.ds(off, _L)]
        rc16 = rc_v[pl.ds(off, _L)]
        r16 = rc16 & 0xFFFF
        k16 = lax.shift_right_logical(rc16, 16)
        xg = plsc.load_gather(x_v, [k16])
        plsc.addupdate_scatter(ax_v, [r16], v16 * xg)
        lg = plsc.load_gather(lam_v, [r16])
        plsc.addupdate_scatter(atl_v, [k16], v16 * lg)

    with jax.named_scope("p1_spmm"):
        plsc.parallel_loop(0, _SUB, _L, unroll=_UNROLL)(nnz_body)
        for cp in cps_rest:
            cp.wait()
        plsc.parallel_loop(_SUB, _CH, _L, unroll=_UNROLL)(nnz_body)

    with jax.named_scope("p2_pub"):
        cp1 = pltpu.async_copy(ax_v, part_ax.at[s], sem_x)
        cp2 = pltpu.async_copy(atl_v, part_atl.at[s], sem_x)
        cp1.wait()
        cp2.wait()
        plsc.subcore_barrier()

    off = j * _SLICE
    cps = [
        pltpu.async_copy(
            part_ax.at[pl.ds(g0, _TPG), pl.ds(off, _SLICE)], bufa_v, sem_x),
        pltpu.async_copy(
            part_atl.at[pl.ds(g0, _TPG), pl.ds(off, _SLICE)], bufb_v, sem_x),
    ]
    with jax.named_scope("p2_pull"):
        for cp in cps:
            cp.wait()

    def loss_body(t, acc):
        acc_p, acc_d, acc_s, acc_c = acc
        ds16 = pl.ds(t * _L, _L)
        ax16 = bufa_v[0, ds16]
        atl16 = bufb_v[0, ds16]
        for k in range(1, _TPG):
            ax16 = ax16 + bufa_v[k, ds16]
            atl16 = atl16 + bufb_v[k, ds16]
        b16 = b_v[ds16]
        c16 = c_v[ds16]
        lam16 = lam_v[pl.ds(off + t * _L, _L)]
        axmb = ax16 - b16
        relu_axmb = jnp.maximum(axmb, 0.0)
        relu_nlam = jnp.maximum(-lam16, 0.0)
        st = atl16 + c16
        cp16 = lam16 * axmb
        return (acc_p + relu_axmb * relu_axmb,
                acc_d + relu_nlam * relu_nlam,
                acc_s + st * st,
                acc_c + cp16 * cp16)

    acc0 = (zero16, zero16, zero16, zero16)
    for cp in cps_bc:
        cp.wait()
    with jax.named_scope("p3_loss"):
        acc_p, acc_d, acc_s, acc_c = lax.fori_loop(
            0, _SLICE // _L, loss_body, acc0)

    scale = 1.0 / (_M * _B)
    outv[...] = (_W_PRIMAL * acc_p + _W_DUAL * acc_d
                 + _W_STAT * acc_s + _W_COMP * acc_c) * scale
    pltpu.async_copy(outv, out_hbm.at[wid], sem_x).wait()


@jax.jit
def _run(x_hat, lam_hat, vals_f, rc_f, b_f, c_f):
    mesh = plsc.VectorSubcoreMesh(core_axis_name="c", subcore_axis_name="s",
                                  num_cores=_NC, num_subcores=_NS)
    kern = pl.kernel(
        _sc_kkt,
        out_type=jax.ShapeDtypeStruct((_NW, _L), jnp.float32),
        mesh=mesh,
        scratch_types=[
            pltpu.VMEM((_CH,), jnp.float32),
            pltpu.VMEM((_CH,), jnp.int32),
            pltpu.VMEM((_N,), jnp.float32),
            pltpu.VMEM((_M,), jnp.float32),
            pltpu.VMEM((_M,), jnp.float32),
            pltpu.VMEM((_N,), jnp.float32),
            pltpu.VMEM((_TPG, _SLICE), jnp.float32),
            pltpu.VMEM((_TPG, _SLICE), jnp.float32),
            pltpu.VMEM((_SLICE,), jnp.float32),
            pltpu.VMEM((_SLICE,), jnp.float32),
            pltpu.VMEM((_L,), jnp.float32),
            pltpu.VMEM_SHARED((_NS, _M), jnp.float32),
            pltpu.VMEM_SHARED((_NS, _N), jnp.float32),
            pltpu.VMEM_SHARED((2, _N), jnp.float32),
            pltpu.VMEM_SHARED((2, _M), jnp.float32),
            pltpu.SemaphoreType.DMA,
            pltpu.SemaphoreType.DMA,
            pltpu.SemaphoreType.DMA,
        ],
        compiler_params=pltpu.CompilerParams(needs_layout_passes=False),
    )
    out = kern(x_hat, lam_hat, vals_f, rc_f, b_f, c_f)
    return jnp.sum(out)


def kernel(x_hat, lam_hat, A_vals, A_rows, A_cols, b_pad, c_pad):
    pad = _NNZ_PAD - _NNZ
    vals_f = jnp.pad(A_vals, ((0, 0), (0, pad))).reshape(-1)
    rc = A_rows.astype(jnp.int32) | (A_cols.astype(jnp.int32) << 16)
    rc_f = jnp.pad(rc, ((0, 0), (0, pad))).reshape(-1)
    return _run(x_hat.astype(jnp.float32), lam_hat.astype(jnp.float32),
                vals_f, rc_f,
                b_pad.reshape(-1).astype(jnp.float32),
                c_pad.reshape(-1).astype(jnp.float32))


# speedup vs baseline: 1.2333x; 1.0029x over previous
import jax
import jax.numpy as jnp
from jax import lax
from jax.experimental import pallas as pl
from jax.experimental.pallas import tpu as pltpu
from jax.experimental.pallas import tpu_sc as plsc

_B, _M, _N = 4, 4096, 4096
_NNZ = 167772
_NC, _NS, _L = 2, 16, 16
_NW = _NC * _NS
_TPG = _NW // _B
_CH = 20992
_NNZ_PAD = _CH * _TPG
_NSPLIT = 4
_SUB = _CH // _NSPLIT
_SLICE = _M // _TPG
_UNROLL = 8

_W_PRIMAL, _W_DUAL, _W_STAT, _W_COMP = 0.1, 0.1, 0.6, 0.2


def _sc_kkt(x_hbm, lam_hbm, vals_hbm, rc_hbm, b_hbm, c_hbm,
            out_hbm,
            vals_v, rc_v, x_v, lam_v, ax_v, atl_v,
            bufa_v, bufb_v, b_v, c_v, outv,
            part_ax, part_atl, sh_x, sh_lam, sem_a, sem_b, sem_x):
    c = lax.axis_index("c")
    s = lax.axis_index("s")
    p = c * 2 + s // _TPG
    j = s % _TPG
    g0 = (s // _TPG) * _TPG
    wid = c * _NS + s
    pi = s // _TPG
    nz_base = p * _NNZ_PAD + j * _CH
    scope = jax.named_scope

    @pl.when(j == 0)
    def _stage_xl():
        pltpu.async_copy(x_hbm.at[pl.ds(p * _N, _N)], sh_x.at[pi], sem_x)
        pltpu.async_copy(lam_hbm.at[pl.ds(p * _M, _M)], sh_lam.at[pi], sem_x)

    cps_bc = [
        pltpu.async_copy(b_hbm.at[pl.ds(p * _M + j * _SLICE, _SLICE)], b_v, sem_x),
        pltpu.async_copy(c_hbm.at[pl.ds(p * _N + j * _SLICE, _SLICE)], c_v, sem_x),
    ]

    def fire(q, csem):
        base = nz_base + q * _SUB
        dst = pl.ds(q * _SUB, _SUB)
        return [
            pltpu.async_copy(vals_hbm.at[pl.ds(base, _SUB)],
                             vals_v.at[dst], csem),
            pltpu.async_copy(rc_hbm.at[pl.ds(base, _SUB)],
                             rc_v.at[dst], csem),
        ]

    cps_q0 = fire(0, sem_a)

    zero16 = jnp.zeros((_L,), jnp.float32)

    def zero_body(off):
        ax_v[pl.ds(off, _L)] = zero16
        atl_v[pl.ds(off, _L)] = zero16

    with jax.named_scope("p0_zero"):
        plsc.parallel_loop(0, _M, _L, unroll=8)(zero_body)

    with jax.named_scope("p0_bcast"):
        @pl.when(j == 0)
        def _wait_xl():
            pltpu.make_async_copy(x_hbm.at[pl.ds(p * _N, _N)],
                                  sh_x.at[pi], sem_x).wait()
            pltpu.make_async_copy(lam_hbm.at[pl.ds(p * _M, _M)],
                                  sh_lam.at[pi], sem_x).wait()
        plsc.subcore_barrier()
        cp1 = pltpu.async_copy(sh_x.at[pi], x_v, sem_x)
        cp2 = pltpu.async_copy(sh_lam.at[pi], lam_v, sem_x)
        cp1.wait()
        cp2.wait()

    with jax.named_scope("p0_wait"):
        for cp in cps_q0:
            cp.wait()

    cps_rest = []
    for q in range(1, _NSPLIT):
        cps_rest += fire(q, sem_b)

    def nnz_body(off):
        v16 = vals_v[pl.ds(off, _L)]
        rc16 = rc_v[pl.ds(off, _L)]
        r16 = rc16 & 0xFFFF
        k16 = lax.shift_right_logical(rc16, 16)
        xg = plsc.load_gather(x_v, [k16])
        plsc.addupdate_scatter(ax_v, [r16], v16 * xg)
        lg = plsc.load_gather(lam_v, [r16])
        plsc.addupdate_scatter(atl_v, [k16], v16 * lg)

    with jax.named_scope("p1_spmm"):
        plsc.parallel_loop(0, _SUB, _L, unroll=_UNROLL)(nnz_body)
        for cp in cps_rest:
            cp.wait()
        plsc.parallel_loop(_SUB, _CH, _L, unroll=_UNROLL)(nnz_body)

    with jax.named_scope("p2_pub"):
        cp1 = pltpu.async_copy(ax_v, part_ax.at[s], sem_x)
        cp2 = pltpu.async_copy(atl_v, part_atl.at[s], sem_x)
        cp1.wait()
        cp2.wait()
        plsc.subcore_barrier()

    off = j * _SLICE
    cps = [
        pltpu.async_copy(
            part_ax.at[pl.ds(g0, _TPG), pl.ds(off, _SLICE)], bufa_v, sem_x),
        pltpu.async_copy(
            part_atl.at[pl.ds(g0, _TPG), pl.ds(off, _SLICE)], bufb_v, sem_x),
    ]
    with jax.named_scope("p2_pull"):
        for cp in cps:
            cp.wait()

    def loss_body(t, acc):
        acc_p, acc_d, acc_s, acc_c = acc
        ds16 = pl.ds(t * _L, _L)
        ax16 = bufa_v[0, ds16]
        atl16 = bufb_v[0, ds16]
        for k in range(1, _TPG):
            ax16 = ax16 + bufa_v[k, ds16]
            atl16 = atl16 + bufb_v[k, ds16]
        b16 = b_v[ds16]
        c16 = c_v[ds16]
        lam16 = lam_v[pl.ds(off + t * _L, _L)]
        axmb = ax16 - b16
        relu_axmb = jnp.maximum(axmb, 0.0)
        relu_nlam = jnp.maximum(-lam16, 0.0)
        st = atl16 + c16
        cp16 = lam16 * axmb
        return (acc_p + relu_axmb * relu_axmb,
                acc_d + relu_nlam * relu_nlam,
                acc_s + st * st,
                acc_c + cp16 * cp16)

    acc0 = (zero16, zero16, zero16, zero16)
    for cp in cps_bc:
        cp.wait()
    with jax.named_scope("p3_loss"):
        acc_p, acc_d, acc_s, acc_c = lax.fori_loop(
            0, _SLICE // _L, loss_body, acc0)

    scale = 1.0 / (_M * _B)
    outv[...] = (_W_PRIMAL * acc_p + _W_DUAL * acc_d
                 + _W_STAT * acc_s + _W_COMP * acc_c) * scale
    pltpu.async_copy(outv, out_hbm.at[pl.ds(wid * _L, _L)], sem_x).wait()


@jax.jit
def _run(x_hat, lam_hat, vals_f, rc_f, b_f, c_f):
    mesh = plsc.VectorSubcoreMesh(core_axis_name="c", subcore_axis_name="s",
                                  num_cores=_NC, num_subcores=_NS)
    kern = pl.kernel(
        _sc_kkt,
        out_type=jax.ShapeDtypeStruct((_NW * _L,), jnp.float32),
        mesh=mesh,
        scratch_types=[
            pltpu.VMEM((_CH,), jnp.float32),
            pltpu.VMEM((_CH,), jnp.int32),
            pltpu.VMEM((_N,), jnp.float32),
            pltpu.VMEM((_M,), jnp.float32),
            pltpu.VMEM((_M,), jnp.float32),
            pltpu.VMEM((_N,), jnp.float32),
            pltpu.VMEM((_TPG, _SLICE), jnp.float32),
            pltpu.VMEM((_TPG, _SLICE), jnp.float32),
            pltpu.VMEM((_SLICE,), jnp.float32),
            pltpu.VMEM((_SLICE,), jnp.float32),
            pltpu.VMEM((_L,), jnp.float32),
            pltpu.VMEM_SHARED((_NS, _M), jnp.float32),
            pltpu.VMEM_SHARED((_NS, _N), jnp.float32),
            pltpu.VMEM_SHARED((2, _N), jnp.float32),
            pltpu.VMEM_SHARED((2, _M), jnp.float32),
            pltpu.SemaphoreType.DMA,
            pltpu.SemaphoreType.DMA,
            pltpu.SemaphoreType.DMA,
        ],
        compiler_params=pltpu.CompilerParams(needs_layout_passes=False),
    )
    out = kern(x_hat, lam_hat, vals_f, rc_f, b_f, c_f)
    return jnp.sum(out)


def kernel(x_hat, lam_hat, A_vals, A_rows, A_cols, b_pad, c_pad):
    pad = _NNZ_PAD - _NNZ
    vals_f = jnp.pad(A_vals, ((0, 0), (0, pad))).reshape(-1)
    rc = A_rows.astype(jnp.int32) | (A_cols.astype(jnp.int32) << 16)
    rc_f = jnp.pad(rc, ((0, 0), (0, pad))).reshape(-1)
    return _run(x_hat.astype(jnp.float32), lam_hat.astype(jnp.float32),
                vals_f, rc_f,
                b_pad.reshape(-1).astype(jnp.float32),
                c_pad.reshape(-1).astype(jnp.float32))
